# Initial kernel scaffold; baseline (speedup 1.0000x reference)
import jax
import jax.numpy as jnp
from jax.experimental import pallas as pl


def kernel(input_ids, table, W, b):
    raise NotImplementedError("write your pallas kernel here")


# speedup vs baseline: 6.4690x; 6.4690x over previous
import jax
import jax.numpy as jnp
from jax import lax
from jax.experimental import pallas as pl
from jax.experimental.pallas import tpu as pltpu
from jax.experimental.pallas import tpu_sc as plsc

NC, NS, L = 2, 16, 16
NW = NC * NS

BATCH, SEQ = 4096, 200
EMBED, HIDDEN = 128, 128
SEQ_HALF = SEQ // 2
ROWS_PER_W = BATCH // NW
NCHUNK = EMBED // L


def _pool_body(ids_hbm, table_hbm, out_hbm, idx_v, rows_v, acc_v, sem):
    wid = lax.axis_index("s") * NC + lax.axis_index("c")

    def row_step(r, carry):
        b = wid * ROWS_PER_W + r
        pltpu.sync_copy(ids_hbm.at[b], idx_v)
        cp0 = pltpu.async_copy(table_hbm.at[idx_v.at[0]], rows_v.at[0], sem)
        cp1 = pltpu.async_copy(table_hbm.at[idx_v.at[1]], rows_v.at[1], sem)
        cp0.wait()
        cp1.wait()

        def seq_step(s, acc):
            acc = tuple(acc[k] + rows_v[0, s, pl.ds(L * k, L)]
                        for k in range(NCHUNK))
            return tuple(acc[k] + rows_v[1, s, pl.ds(L * k, L)]
                         for k in range(NCHUNK))

        acc = lax.fori_loop(
            0, SEQ_HALF, seq_step,
            tuple(jnp.zeros((L,), jnp.float32) for _ in range(NCHUNK)))
        for k in range(NCHUNK):
            acc_v[pl.ds(L * k, L)] = acc[k] * (1.0 / SEQ)
        pltpu.sync_copy(acc_v, out_hbm.at[b])
        return carry

    lax.fori_loop(0, ROWS_PER_W, row_step, 0)


_pool = pl.kernel(
    _pool_body,
    out_type=jax.ShapeDtypeStruct((BATCH, EMBED), jnp.float32),
    mesh=plsc.VectorSubcoreMesh(core_axis_name="c", subcore_axis_name="s",
                                num_cores=NC, num_subcores=NS),
    scratch_types=[
        pltpu.VMEM((2, SEQ_HALF), jnp.int32),
        pltpu.VMEM((2, SEQ_HALF, EMBED), jnp.float32),
        pltpu.VMEM((EMBED,), jnp.float32),
        pltpu.SemaphoreType.DMA,
    ],
)


def _linear_body(p_ref, w_ref, b_ref, o_ref):
    o_ref[...] = lax.dot_general(
        p_ref[...], w_ref[...], (((1,), (1,)), ((), ())),
        preferred_element_type=jnp.float32) + b_ref[...]


def kernel(input_ids, table, W, b):
    ids2 = input_ids.astype(jnp.int32).reshape(BATCH, 2, SEQ_HALF)
    pooled = _pool(ids2, table)
    out = pl.pallas_call(
        _linear_body,
        out_shape=jax.ShapeDtypeStruct((BATCH, HIDDEN), jnp.float32),
        grid=(BATCH // 1024,),
        in_specs=[
            pl.BlockSpec((1024, EMBED), lambda i: (i, 0)),
            pl.BlockSpec((HIDDEN, EMBED), lambda i: (0, 0)),
            pl.BlockSpec((1, HIDDEN), lambda i: (0, 0)),
        ],
        out_specs=pl.BlockSpec((1024, HIDDEN), lambda i: (i, 0)),
    )(pooled, W, b.reshape(1, HIDDEN))
    return out


# speedup vs baseline: 11.4627x; 1.7720x over previous
import jax
import jax.numpy as jnp
from jax import lax
from jax.experimental import pallas as pl
from jax.experimental.pallas import tpu as pltpu
from jax.experimental.pallas import tpu_sc as plsc

NC, NS, L = 2, 16, 16
NW = NC * NS

BATCH, SEQ = 4096, 200
EMBED, HIDDEN = 128, 128
SEQ_HALF = SEQ // 2
ROWS_PER_W = BATCH // NW
NCHUNK = EMBED // L


def _pool_body(ids_hbm, table_hbm, out_hbm, idx_v, rows_v, acc_v, sem0, sem1):
    wid = lax.axis_index("s") * NC + lax.axis_index("c")
    base = wid * ROWS_PER_W
    sems = (sem0, sem1)

    def issue(slot, b):
        pltpu.sync_copy(ids_hbm.at[b], idx_v.at[slot])
        for j in range(2):
            pltpu.async_copy(table_hbm.at[idx_v.at[slot, j]],
                             rows_v.at[slot, j], sems[slot])

    def wait(slot):
        for j in range(2):
            pltpu.make_async_copy(table_hbm.at[idx_v.at[slot, j]],
                                  rows_v.at[slot, j], sems[slot]).wait()

    def reduce_store(slot, b):
        def seq_step(s, acc):
            acc = tuple(acc[k] + rows_v[slot, 0, s, pl.ds(L * k, L)]
                        for k in range(NCHUNK))
            return tuple(acc[k] + rows_v[slot, 1, s, pl.ds(L * k, L)]
                         for k in range(NCHUNK))

        acc = lax.fori_loop(
            0, SEQ_HALF, seq_step,
            tuple(jnp.zeros((L,), jnp.float32) for _ in range(NCHUNK)))
        for k in range(NCHUNK):
            acc_v[pl.ds(L * k, L)] = acc[k] * (1.0 / SEQ)
        pltpu.sync_copy(acc_v, out_hbm.at[b])

    issue(0, base)

    def pair_step(i, carry):
        b0 = base + 2 * i
        issue(1, b0 + 1)
        wait(0)
        reduce_store(0, b0)

        @pl.when(2 * i + 2 < ROWS_PER_W)
        def _():
            issue(0, b0 + 2)

        wait(1)
        reduce_store(1, b0 + 1)
        return carry

    lax.fori_loop(0, ROWS_PER_W // 2, pair_step, 0)


_pool = pl.kernel(
    _pool_body,
    out_type=jax.ShapeDtypeStruct((BATCH, EMBED), jnp.float32),
    mesh=plsc.VectorSubcoreMesh(core_axis_name="c", subcore_axis_name="s",
                                num_cores=NC, num_subcores=NS),
    scratch_types=[
        pltpu.VMEM((2, 2, SEQ_HALF), jnp.int32),
        pltpu.VMEM((2, 2, SEQ_HALF, EMBED), jnp.float32),
        pltpu.VMEM((EMBED,), jnp.float32),
        pltpu.SemaphoreType.DMA,
        pltpu.SemaphoreType.DMA,
    ],
)


def _linear_body(p_ref, w_ref, b_ref, o_ref):
    o_ref[...] = lax.dot_general(
        p_ref[...], w_ref[...], (((1,), (1,)), ((), ())),
        preferred_element_type=jnp.float32) + b_ref[...]


def kernel(input_ids, table, W, b):
    ids2 = input_ids.astype(jnp.int32).reshape(BATCH, 2, SEQ_HALF)
    pooled = _pool(ids2, table)
    out = pl.pallas_call(
        _linear_body,
        out_shape=jax.ShapeDtypeStruct((BATCH, HIDDEN), jnp.float32),
        grid=(BATCH // 1024,),
        in_specs=[
            pl.BlockSpec((1024, EMBED), lambda i: (i, 0)),
            pl.BlockSpec((HIDDEN, EMBED), lambda i: (0, 0)),
            pl.BlockSpec((1, HIDDEN), lambda i: (0, 0)),
        ],
        out_specs=pl.BlockSpec((1024, HIDDEN), lambda i: (i, 0)),
    )(pooled, W, b.reshape(1, HIDDEN))
    return out


# speedup vs baseline: 13.4385x; 1.1724x over previous
import jax
import jax.numpy as jnp
from jax import lax
from jax.experimental import pallas as pl
from jax.experimental.pallas import tpu as pltpu
from jax.experimental.pallas import tpu_sc as plsc

NC, NS, L = 2, 16, 16
NW = NC * NS

BATCH, SEQ = 4096, 200
EMBED, HIDDEN = 128, 128
SEQ_HALF = SEQ // 2
ROWS_PER_W = BATCH // NW
NCHUNK = EMBED // L


def _pool_body(ids_hbm, table_hbm, out_hbm, idx_all, rows_v, acc_v,
               sem0, sem1, osem0, osem1):
    wid = lax.axis_index("s") * NC + lax.axis_index("c")
    base = wid * ROWS_PER_W
    sems = (sem0, sem1)
    osems = (osem0, osem1)

    pltpu.sync_copy(ids_hbm.at[pl.ds(base, ROWS_PER_W)], idx_all)

    def issue(slot, r):
        for j in range(2):
            pltpu.async_copy(table_hbm.at[idx_all.at[r, j]],
                             rows_v.at[slot, j], sems[slot])

    def wait(slot, r):
        for j in range(2):
            pltpu.make_async_copy(table_hbm.at[idx_all.at[r, j]],
                                  rows_v.at[slot, j], sems[slot]).wait()

    def reduce_store(slot, r, i):
        def seq_step(s, acc):
            acc = tuple(acc[k] + rows_v[slot, 0, s, pl.ds(L * k, L)]
                        for k in range(NCHUNK))
            return tuple(acc[k] + rows_v[slot, 1, s, pl.ds(L * k, L)]
                         for k in range(NCHUNK))

        acc = lax.fori_loop(
            0, SEQ_HALF, seq_step,
            tuple(jnp.zeros((L,), jnp.float32) for _ in range(NCHUNK)))

        @pl.when(i > 0)
        def _():
            pltpu.make_async_copy(acc_v.at[slot], out_hbm.at[base + r],
                                  osems[slot]).wait()

        for k in range(NCHUNK):
            acc_v[slot, pl.ds(L * k, L)] = acc[k] * (1.0 / SEQ)
        pltpu.async_copy(acc_v.at[slot], out_hbm.at[base + r], osems[slot])

    issue(0, 0)

    def pair_step(i, carry):
        r0 = 2 * i
        issue(1, r0 + 1)
        wait(0, r0)
        reduce_store(0, r0, i)

        @pl.when(r0 + 2 < ROWS_PER_W)
        def _():
            issue(0, r0 + 2)

        wait(1, r0 + 1)
        reduce_store(1, r0 + 1, i)
        return carry

    lax.fori_loop(0, ROWS_PER_W // 2, pair_step, 0)

    last = ROWS_PER_W - 2
    for slot in range(2):
        pltpu.make_async_copy(acc_v.at[slot], out_hbm.at[base + last + slot],
                              osems[slot]).wait()


_pool = pl.kernel(
    _pool_body,
    out_type=jax.ShapeDtypeStruct((BATCH, EMBED), jnp.float32),
    mesh=plsc.VectorSubcoreMesh(core_axis_name="c", subcore_axis_name="s",
                                num_cores=NC, num_subcores=NS),
    scratch_types=[
        pltpu.VMEM((ROWS_PER_W, 2, SEQ_HALF), jnp.int32),
        pltpu.VMEM((2, 2, SEQ_HALF, EMBED), jnp.float32),
        pltpu.VMEM((2, EMBED), jnp.float32),
        pltpu.SemaphoreType.DMA,
        pltpu.SemaphoreType.DMA,
        pltpu.SemaphoreType.DMA,
        pltpu.SemaphoreType.DMA,
    ],
)


def _linear_body(p_ref, w_ref, b_ref, o_ref):
    o_ref[...] = lax.dot_general(
        p_ref[...], w_ref[...], (((1,), (1,)), ((), ())),
        preferred_element_type=jnp.float32) + b_ref[...]


def kernel(input_ids, table, W, b):
    ids2 = input_ids.astype(jnp.int32).reshape(BATCH, 2, SEQ_HALF)
    pooled = _pool(ids2, table)
    out = pl.pallas_call(
        _linear_body,
        out_shape=jax.ShapeDtypeStruct((BATCH, HIDDEN), jnp.float32),
        grid=(BATCH // 1024,),
        in_specs=[
            pl.BlockSpec((1024, EMBED), lambda i: (i, 0)),
            pl.BlockSpec((HIDDEN, EMBED), lambda i: (0, 0)),
            pl.BlockSpec((1, HIDDEN), lambda i: (0, 0)),
        ],
        out_specs=pl.BlockSpec((1024, HIDDEN), lambda i: (i, 0)),
    )(pooled, W, b.reshape(1, HIDDEN))
    return out
